# baseline (speedup 1.0000x reference)
import jax
import jax.numpy as jnp
from jax import lax
from jax.experimental import pallas as pl


def _row_pool(X, n):
    ii = lax.broadcasted_iota(jnp.int32, (n, n), 0)
    jj = lax.broadcasted_iota(jnp.int32, (n, n), 1)
    m = (ii < jj)[None, :, :]
    a = X[:, :, None]
    b = X[:, None, :]
    cmpb = (a < b) | ((a <= b) & m)
    rank = jnp.sum(cmpb.astype(jnp.int32), axis=1)
    binv = lax.shift_right_logical(rank, 1)
    w3 = lax.broadcasted_iota(jnp.int32, (1, 1, n // 2), 2)
    jv = lax.broadcasted_iota(jnp.int32, (1, n, 1), 1)
    contrib = jnp.where(binv[:, :, None] == w3, jv, 0)
    return jnp.sum(contrib, axis=1).astype(jnp.float32)


def _body(x_ref, o_ref):
    X = x_ref[...]
    n = X.shape[-1]
    p0 = _row_pool(X[:, 0, :], n)
    p1 = _row_pool(X[:, 1, :], n)
    o_ref[...] = (p0 + p1) * 0.25


def kernel(x):
    B, C, H, W = x.shape
    pairs = (B * C * H) // 2
    xr = x.reshape(pairs, 2, W)
    RP = 8
    out = pl.pallas_call(
        _body,
        grid=(pairs // RP,),
        in_specs=[pl.BlockSpec((RP, 2, W), lambda g: (g, 0, 0))],
        out_specs=pl.BlockSpec((RP, W // 2), lambda g: (g, 0)),
        out_shape=jax.ShapeDtypeStruct((pairs, W // 2), jnp.float32),
    )(xr)
    return out.reshape(B, C, H // 2, W // 2)
